# Initial kernel scaffold; baseline (speedup 1.0000x reference)
import jax
import jax.numpy as jnp
from jax.experimental import pallas as pl


def kernel(data, s, mu_W1, mu_b1, mu_W2, mu_b2, lv_W1, lv_b1, lv_W2, lv_b2, dec_W1, dec_b1, dec_W2, dec_b2):
    raise NotImplementedError("write your pallas kernel here")


# speedup vs baseline: 1.0872x; 1.0872x over previous
import functools

import jax
import jax.numpy as jnp
from jax.experimental import pallas as pl
from jax.experimental.pallas import tpu as pltpu


def _lrelu(x):
    return jnp.where(x >= 0, x, 0.01 * x)


def _vae_body(x_ref, s_ref, muW1_ref, mub1_ref, muW2_ref, mub2_ref,
              lvW1_ref, lvb1_ref, lvW2_ref, lvb2_ref,
              dW1_ref, drow_ref, db1_ref, dW2_ref, db2_ref,
              rec_ref, mu_ref, lv_ref):
    x = x_ref[...]
    f32 = jnp.float32

    h = jnp.dot(x, muW1_ref[...], preferred_element_type=f32) + mub1_ref[...]
    h = _lrelu(h).astype(jnp.bfloat16)
    mu = jnp.dot(h, muW2_ref[...], preferred_element_type=f32) + mub2_ref[...]
    mu_ref[...] = mu

    g = jnp.dot(x, lvW1_ref[...], preferred_element_type=f32) + lvb1_ref[...]
    g = _lrelu(g).astype(jnp.bfloat16)
    lv_ref[...] = jnp.dot(g, lvW2_ref[...], preferred_element_type=f32) + lvb2_ref[...]

    t = jnp.dot(mu.astype(jnp.bfloat16), dW1_ref[...], preferred_element_type=f32)
    t = t + s_ref[...] * drow_ref[...] + db1_ref[...]
    t = _lrelu(t).astype(jnp.bfloat16)
    rec_ref[...] = jnp.dot(t, dW2_ref[...], preferred_element_type=f32) + db2_ref[...]


@functools.partial(jax.jit, static_argnames=())
def kernel(data, s, mu_W1, mu_b1, mu_W2, mu_b2, lv_W1, lv_b1, lv_W2, lv_b2,
           dec_W1, dec_b1, dec_W2, dec_b2):
    n, dx = data.shape
    dh = mu_W1.shape[1]
    bn = 128 if n % 128 == 0 else n

    bf16 = jnp.bfloat16
    x16 = data.astype(bf16)
    dW1_main = dec_W1[:dh].astype(bf16)
    drow = dec_W1[dh:dh + 1]

    row_blk = lambda i: (i, 0)
    whole = lambda i: (0, 0)
    vec = lambda i: (0,)

    grid = (n // bn,)
    rec, mu, lv = pl.pallas_call(
        _vae_body,
        grid=grid,
        in_specs=[
            pl.BlockSpec((bn, dx), row_blk),
            pl.BlockSpec((bn, 1), row_blk),
            pl.BlockSpec((dx, dh), whole),
            pl.BlockSpec((dh,), vec),
            pl.BlockSpec((dh, dh), whole),
            pl.BlockSpec((dh,), vec),
            pl.BlockSpec((dx, dh), whole),
            pl.BlockSpec((dh,), vec),
            pl.BlockSpec((dh, dh), whole),
            pl.BlockSpec((dh,), vec),
            pl.BlockSpec((dh, dh), whole),
            pl.BlockSpec((1, dh), whole),
            pl.BlockSpec((dh,), vec),
            pl.BlockSpec((dh, dx), whole),
            pl.BlockSpec((dx,), vec),
        ],
        out_specs=[
            pl.BlockSpec((bn, dx), row_blk),
            pl.BlockSpec((bn, dh), row_blk),
            pl.BlockSpec((bn, dh), row_blk),
        ],
        out_shape=[
            jax.ShapeDtypeStruct((n, dx), jnp.float32),
            jax.ShapeDtypeStruct((n, dh), jnp.float32),
            jax.ShapeDtypeStruct((n, dh), jnp.float32),
        ],
        compiler_params=pltpu.CompilerParams(
            dimension_semantics=("arbitrary",),
        ),
    )(x16, s,
      mu_W1.astype(bf16), mu_b1, mu_W2.astype(bf16), mu_b2,
      lv_W1.astype(bf16), lv_b1, lv_W2.astype(bf16), lv_b2,
      dW1_main, drow, dec_b1, dec_W2.astype(bf16), dec_b2)

    return (rec, mu, lv, mu)


# speedup vs baseline: 1.1709x; 1.0770x over previous
import jax
import jax.numpy as jnp
from jax.experimental import pallas as pl
from jax.experimental.pallas import tpu as pltpu


def _lrelu(x):
    return jnp.where(x >= 0, x, 0.01 * x)


def _mu_dec_body(x_ref, s_ref, muW1_ref, mub1_ref, muW2_ref, mub2_ref,
                 dW1_ref, drow_ref, db1_ref, dW2_ref, db2_ref,
                 rec_ref, mu_ref):
    f32 = jnp.float32
    bf16 = jnp.bfloat16
    x = x_ref[...].astype(bf16)
    h = jnp.dot(x, muW1_ref[...], preferred_element_type=f32) + mub1_ref[...]
    h = _lrelu(h).astype(bf16)
    mu = jnp.dot(h, muW2_ref[...], preferred_element_type=f32) + mub2_ref[...]
    mu_ref[...] = mu
    t = jnp.dot(mu.astype(bf16), dW1_ref[...], preferred_element_type=f32)
    t = t + s_ref[...] * drow_ref[...] + db1_ref[...]
    t = _lrelu(t).astype(bf16)
    rec_ref[...] = jnp.dot(t, dW2_ref[...], preferred_element_type=f32) + db2_ref[...]


def _lv_body(x_ref, lvW1_ref, lvb1_ref, lvW2_ref, lvb2_ref, lv_ref):
    f32 = jnp.float32
    bf16 = jnp.bfloat16
    x = x_ref[...].astype(bf16)
    g = jnp.dot(x, lvW1_ref[...], preferred_element_type=f32) + lvb1_ref[...]
    g = _lrelu(g).astype(bf16)
    lv_ref[...] = jnp.dot(g, lvW2_ref[...], preferred_element_type=f32) + lvb2_ref[...]


def kernel(data, s, mu_W1, mu_b1, mu_W2, mu_b2, lv_W1, lv_b1, lv_W2, lv_b2,
           dec_W1, dec_b1, dec_W2, dec_b2):
    n, dx = data.shape
    dh = mu_W1.shape[1]
    bn_a = 256 if n % 256 == 0 else n
    bn_b = 512 if n % 512 == 0 else n

    bf16 = jnp.bfloat16
    dW1_main = dec_W1[:dh].astype(bf16)
    drow = dec_W1[dh:dh + 1]

    row_blk = lambda i: (i, 0)
    whole = lambda i: (0, 0)
    vec = lambda i: (0,)

    rec, mu = pl.pallas_call(
        _mu_dec_body,
        grid=(n // bn_a,),
        in_specs=[
            pl.BlockSpec((bn_a, dx), row_blk),
            pl.BlockSpec((bn_a, 1), row_blk),
            pl.BlockSpec((dx, dh), whole),
            pl.BlockSpec((dh,), vec),
            pl.BlockSpec((dh, dh), whole),
            pl.BlockSpec((dh,), vec),
            pl.BlockSpec((dh, dh), whole),
            pl.BlockSpec((1, dh), whole),
            pl.BlockSpec((dh,), vec),
            pl.BlockSpec((dh, dx), whole),
            pl.BlockSpec((dx,), vec),
        ],
        out_specs=[
            pl.BlockSpec((bn_a, dx), row_blk),
            pl.BlockSpec((bn_a, dh), row_blk),
        ],
        out_shape=[
            jax.ShapeDtypeStruct((n, dx), jnp.float32),
            jax.ShapeDtypeStruct((n, dh), jnp.float32),
        ],
        compiler_params=pltpu.CompilerParams(
            dimension_semantics=("arbitrary",),
        ),
    )(data, s,
      mu_W1.astype(bf16), mu_b1, mu_W2.astype(bf16), mu_b2,
      dW1_main, drow, dec_b1, dec_W2.astype(bf16), dec_b2)

    lv = pl.pallas_call(
        _lv_body,
        grid=(n // bn_b,),
        in_specs=[
            pl.BlockSpec((bn_b, dx), row_blk),
            pl.BlockSpec((dx, dh), whole),
            pl.BlockSpec((dh,), vec),
            pl.BlockSpec((dh, dh), whole),
            pl.BlockSpec((dh,), vec),
        ],
        out_specs=pl.BlockSpec((bn_b, dh), row_blk),
        out_shape=jax.ShapeDtypeStruct((n, dh), jnp.float32),
        compiler_params=pltpu.CompilerParams(
            dimension_semantics=("arbitrary",),
        ),
    )(data, lv_W1.astype(bf16), lv_b1, lv_W2.astype(bf16), lv_b2)

    return (rec, mu, lv, mu)


# speedup vs baseline: 1.1734x; 1.0021x over previous
import jax
import jax.numpy as jnp
from jax.experimental import pallas as pl
from jax.experimental.pallas import tpu as pltpu


def _lrelu(x):
    return jnp.where(x >= 0, x, 0.01 * x)


def _mu_dec_body(x_ref, s_ref, muW1_ref, mub1_ref, muW2_ref, mub2_ref,
                 dW1_ref, drow_ref, db1_ref, dW2_ref, db2_ref,
                 rec_ref, mu_ref):
    f32 = jnp.float32
    bf16 = jnp.bfloat16
    x = x_ref[...].astype(bf16)
    h = jnp.dot(x, muW1_ref[...], preferred_element_type=f32) + mub1_ref[...]
    h = _lrelu(h).astype(bf16)
    mu = jnp.dot(h, muW2_ref[...], preferred_element_type=f32) + mub2_ref[...]
    mu_ref[...] = mu
    t = jnp.dot(mu.astype(bf16), dW1_ref[...], preferred_element_type=f32)
    t = t + s_ref[...] * drow_ref[...] + db1_ref[...]
    t = _lrelu(t).astype(bf16)
    rec_ref[...] = jnp.dot(t, dW2_ref[...], preferred_element_type=f32) + db2_ref[...]


def _lv_body(x_ref, lvW1_ref, lvb1_ref, lvW2_ref, lvb2_ref, lv_ref):
    f32 = jnp.float32
    bf16 = jnp.bfloat16
    x = x_ref[...].astype(bf16)
    g = jnp.dot(x, lvW1_ref[...], preferred_element_type=f32) + lvb1_ref[...]
    g = _lrelu(g).astype(bf16)
    lv_ref[...] = jnp.dot(g, lvW2_ref[...], preferred_element_type=f32) + lvb2_ref[...]


def kernel(data, s, mu_W1, mu_b1, mu_W2, mu_b2, lv_W1, lv_b1, lv_W2, lv_b2,
           dec_W1, dec_b1, dec_W2, dec_b2):
    n, dx = data.shape
    dh = mu_W1.shape[1]
    bn_a = 256 if n % 256 == 0 else n
    bn_b = 512 if n % 512 == 0 else n

    bf16 = jnp.bfloat16
    dW1_main = dec_W1[:dh].astype(bf16)
    drow = dec_W1[dh:dh + 1]

    row_blk = lambda i: (i, 0)
    whole = lambda i: (0, 0)
    vec = lambda i: (0,)

    rec, mu = pl.pallas_call(
        _mu_dec_body,
        grid=(n // bn_a,),
        in_specs=[
            pl.BlockSpec((bn_a, dx), row_blk),
            pl.BlockSpec((bn_a, 1), row_blk),
            pl.BlockSpec((dx, dh), whole),
            pl.BlockSpec((dh,), vec),
            pl.BlockSpec((dh, dh), whole),
            pl.BlockSpec((dh,), vec),
            pl.BlockSpec((dh, dh), whole),
            pl.BlockSpec((1, dh), whole),
            pl.BlockSpec((dh,), vec),
            pl.BlockSpec((dh, dx), whole),
            pl.BlockSpec((dx,), vec),
        ],
        out_specs=[
            pl.BlockSpec((bn_a, dx), row_blk),
            pl.BlockSpec((bn_a, dh), row_blk),
        ],
        out_shape=[
            jax.ShapeDtypeStruct((n, dx), jnp.float32),
            jax.ShapeDtypeStruct((n, dh), jnp.float32),
        ],
        compiler_params=pltpu.CompilerParams(
            dimension_semantics=("parallel",),
        ),
    )(data, s,
      mu_W1.astype(bf16), mu_b1, mu_W2.astype(bf16), mu_b2,
      dW1_main, drow, dec_b1, dec_W2.astype(bf16), dec_b2)

    lv = pl.pallas_call(
        _lv_body,
        grid=(n // bn_b,),
        in_specs=[
            pl.BlockSpec((bn_b, dx), row_blk),
            pl.BlockSpec((dx, dh), whole),
            pl.BlockSpec((dh,), vec),
            pl.BlockSpec((dh, dh), whole),
            pl.BlockSpec((dh,), vec),
        ],
        out_specs=pl.BlockSpec((bn_b, dh), row_blk),
        out_shape=jax.ShapeDtypeStruct((n, dh), jnp.float32),
        compiler_params=pltpu.CompilerParams(
            dimension_semantics=("parallel",),
        ),
    )(data, lv_W1.astype(bf16), lv_b1, lv_W2.astype(bf16), lv_b2)

    return (rec, mu, lv, mu)


# speedup vs baseline: 1.2798x; 1.0907x over previous
import jax
import jax.numpy as jnp
from jax.experimental import pallas as pl
from jax.experimental.pallas import tpu as pltpu


def _lrelu(x):
    return jnp.maximum(x, 0.01 * x)


def _mu_dec_body(x_ref, s_ref, muW1_ref, mub1_ref, muW2_ref, mub2_ref,
                 dW1_ref, drow_ref, db1_ref, dW2_ref, db2_ref,
                 rec_ref, mu_ref, mu2_ref):
    f32 = jnp.float32
    bf16 = jnp.bfloat16
    x = x_ref[...].astype(bf16)
    h = jnp.dot(x, muW1_ref[...], preferred_element_type=f32) + mub1_ref[...]
    h = _lrelu(h).astype(bf16)
    mu = jnp.dot(h, muW2_ref[...], preferred_element_type=f32) + mub2_ref[...]
    mu_ref[...] = mu
    mu2_ref[...] = mu
    t = jnp.dot(mu.astype(bf16), dW1_ref[...], preferred_element_type=f32)
    t = t + s_ref[...] * drow_ref[...] + db1_ref[...]
    t = _lrelu(t).astype(bf16)
    rec_ref[...] = jnp.dot(t, dW2_ref[...], preferred_element_type=f32) + db2_ref[...]


def _lv_body(x_ref, lvW1_ref, lvb1_ref, lvW2_ref, lvb2_ref, lv_ref):
    f32 = jnp.float32
    bf16 = jnp.bfloat16
    x = x_ref[...].astype(bf16)
    g = jnp.dot(x, lvW1_ref[...], preferred_element_type=f32) + lvb1_ref[...]
    g = _lrelu(g).astype(bf16)
    lv_ref[...] = jnp.dot(g, lvW2_ref[...], preferred_element_type=f32) + lvb2_ref[...]


def kernel(data, s, mu_W1, mu_b1, mu_W2, mu_b2, lv_W1, lv_b1, lv_W2, lv_b2,
           dec_W1, dec_b1, dec_W2, dec_b2):
    n, dx = data.shape
    dh = mu_W1.shape[1]
    bn_a = 256 if n % 256 == 0 else n
    bn_b = 512 if n % 512 == 0 else n

    bf16 = jnp.bfloat16
    dW1_main = dec_W1[:dh].astype(bf16)
    drow = dec_W1[dh:dh + 1]

    row_blk = lambda i: (i, 0)
    whole = lambda i: (0, 0)
    vec = lambda i: (0,)

    rec, mu, mu2 = pl.pallas_call(
        _mu_dec_body,
        grid=(n // bn_a,),
        in_specs=[
            pl.BlockSpec((bn_a, dx), row_blk),
            pl.BlockSpec((bn_a, 1), row_blk),
            pl.BlockSpec((dx, dh), whole),
            pl.BlockSpec((dh,), vec),
            pl.BlockSpec((dh, dh), whole),
            pl.BlockSpec((dh,), vec),
            pl.BlockSpec((dh, dh), whole),
            pl.BlockSpec((1, dh), whole),
            pl.BlockSpec((dh,), vec),
            pl.BlockSpec((dh, dx), whole),
            pl.BlockSpec((dx,), vec),
        ],
        out_specs=[
            pl.BlockSpec((bn_a, dx), row_blk),
            pl.BlockSpec((bn_a, dh), row_blk),
            pl.BlockSpec((bn_a, dh), row_blk),
        ],
        out_shape=[
            jax.ShapeDtypeStruct((n, dx), jnp.float32),
            jax.ShapeDtypeStruct((n, dh), jnp.float32),
            jax.ShapeDtypeStruct((n, dh), jnp.float32),
        ],
        compiler_params=pltpu.CompilerParams(
            dimension_semantics=("arbitrary",),
        ),
    )(data, s,
      mu_W1.astype(bf16), mu_b1, mu_W2.astype(bf16), mu_b2,
      dW1_main, drow, dec_b1, dec_W2.astype(bf16), dec_b2)

    lv = pl.pallas_call(
        _lv_body,
        grid=(n // bn_b,),
        in_specs=[
            pl.BlockSpec((bn_b, dx), row_blk),
            pl.BlockSpec((dx, dh), whole),
            pl.BlockSpec((dh,), vec),
            pl.BlockSpec((dh, dh), whole),
            pl.BlockSpec((dh,), vec),
        ],
        out_specs=pl.BlockSpec((bn_b, dh), row_blk),
        out_shape=jax.ShapeDtypeStruct((n, dh), jnp.float32),
        compiler_params=pltpu.CompilerParams(
            dimension_semantics=("arbitrary",),
        ),
    )(data, lv_W1.astype(bf16), lv_b1, lv_W2.astype(bf16), lv_b2)

    return (rec, mu, lv, mu2)


# speedup vs baseline: 1.3288x; 1.0382x over previous
import jax
import jax.numpy as jnp
from jax.experimental import pallas as pl
from jax.experimental.pallas import tpu as pltpu


def _lrelu(x):
    return jnp.maximum(x, 0.01 * x)


def _mu_dec_body(x_ref, s_ref, muW1_ref, mub1_ref, muW2_ref, mub2_ref,
                 dW1_ref, drow_ref, db1_ref, dW2_ref, db2_ref,
                 rec_ref, mu_ref, mu2_ref):
    f32 = jnp.float32
    bf16 = jnp.bfloat16
    x = x_ref[...].astype(bf16)
    h = jnp.dot(x, muW1_ref[...].astype(bf16), preferred_element_type=f32) + mub1_ref[...]
    h = _lrelu(h).astype(bf16)
    mu = jnp.dot(h, muW2_ref[...], preferred_element_type=f32) + mub2_ref[...]
    mu_ref[...] = mu
    mu2_ref[...] = mu
    t = jnp.dot(mu.astype(bf16), dW1_ref[...], preferred_element_type=f32)
    t = t + s_ref[...] * drow_ref[...] + db1_ref[...]
    t = _lrelu(t).astype(bf16)
    rec_ref[...] = jnp.dot(t, dW2_ref[...], preferred_element_type=f32) + db2_ref[...]


def _lv_body(x_ref, lvW1_ref, lvb1_ref, lvW2_ref, lvb2_ref, lv_ref):
    f32 = jnp.float32
    bf16 = jnp.bfloat16
    x = x_ref[...].astype(bf16)
    g = jnp.dot(x, lvW1_ref[...].astype(bf16), preferred_element_type=f32) + lvb1_ref[...]
    g = _lrelu(g).astype(bf16)
    lv_ref[...] = jnp.dot(g, lvW2_ref[...].astype(bf16), preferred_element_type=f32) + lvb2_ref[...]


def kernel(data, s, mu_W1, mu_b1, mu_W2, mu_b2, lv_W1, lv_b1, lv_W2, lv_b2,
           dec_W1, dec_b1, dec_W2, dec_b2):
    n, dx = data.shape
    dh = mu_W1.shape[1]
    bn_a = 256 if n % 256 == 0 else n
    bn_b = 512 if n % 512 == 0 else n

    bf16 = jnp.bfloat16
    dW1_main = dec_W1[:dh].astype(bf16)
    drow = dec_W1[dh:dh + 1]

    row_blk = lambda i: (i, 0)
    whole = lambda i: (0, 0)
    vec = lambda i: (0,)

    rec, mu, mu2 = pl.pallas_call(
        _mu_dec_body,
        grid=(n // bn_a,),
        in_specs=[
            pl.BlockSpec((bn_a, dx), row_blk),
            pl.BlockSpec((bn_a, 1), row_blk),
            pl.BlockSpec((dx, dh), whole),
            pl.BlockSpec((dh,), vec),
            pl.BlockSpec((dh, dh), whole),
            pl.BlockSpec((dh,), vec),
            pl.BlockSpec((dh, dh), whole),
            pl.BlockSpec((1, dh), whole),
            pl.BlockSpec((dh,), vec),
            pl.BlockSpec((dh, dx), whole),
            pl.BlockSpec((dx,), vec),
        ],
        out_specs=[
            pl.BlockSpec((bn_a, dx), row_blk),
            pl.BlockSpec((bn_a, dh), row_blk),
            pl.BlockSpec((bn_a, dh), row_blk),
        ],
        out_shape=[
            jax.ShapeDtypeStruct((n, dx), jnp.float32),
            jax.ShapeDtypeStruct((n, dh), jnp.float32),
            jax.ShapeDtypeStruct((n, dh), jnp.float32),
        ],
        compiler_params=pltpu.CompilerParams(
            dimension_semantics=("arbitrary",),
        ),
    )(data, s,
      mu_W1, mu_b1, mu_W2.astype(bf16), mu_b2,
      dW1_main, drow, dec_b1, dec_W2.astype(bf16), dec_b2)

    lv = pl.pallas_call(
        _lv_body,
        grid=(n // bn_b,),
        in_specs=[
            pl.BlockSpec((bn_b, dx), row_blk),
            pl.BlockSpec((dx, dh), whole),
            pl.BlockSpec((dh,), vec),
            pl.BlockSpec((dh, dh), whole),
            pl.BlockSpec((dh,), vec),
        ],
        out_specs=pl.BlockSpec((bn_b, dh), row_blk),
        out_shape=jax.ShapeDtypeStruct((n, dh), jnp.float32),
        compiler_params=pltpu.CompilerParams(
            dimension_semantics=("arbitrary",),
        ),
    )(data, lv_W1, lv_b1, lv_W2, lv_b2)

    return (rec, mu, lv, mu2)
